# baseline (speedup 1.0000x reference)
import jax
import jax.numpy as jnp
from jax import lax
from jax.experimental import pallas as pl

_LR, _B1, _B2, _EPS = 1e-3, 0.9, 0.999, 1e-8
_STEPS = 10
_K = 3


def _train_body(sup_ref, supt_ref, lab_col_ref, lab_row_ref,
                w2t_ref, dsupt_ref, s2_ref):
    sup = sup_ref[...]
    supt = supt_ref[...]
    n, d = sup.shape
    isn = lax.rsqrt(jnp.sum(sup * sup, axis=1, keepdims=True))
    sn = sup * isn
    isn_row = lax.rsqrt(jnp.sum(supt * supt, axis=0, keepdims=True))
    snt = supt * isn_row

    lc = lab_col_ref[...]
    lr = lab_row_ref[...]
    ncls = 64
    oh = jnp.where(
        lc == lax.broadcasted_iota(jnp.int32, (n, ncls), 1), 1.0, 0.0
    ).astype(jnp.float32)
    oht = jnp.where(
        lr == lax.broadcasted_iota(jnp.int32, (ncls, n), 0), 1.0, 0.0
    ).astype(jnp.float32)
    counts = jnp.sum(oh, axis=0, keepdims=True)
    s_all = jnp.float32(n) * jnp.float32(n) - 2.0 * jnp.sum(counts * counts)
    inv_ssum = 2.0 / (s_all + jnp.float32(n))

    wr = lax.broadcasted_iota(jnp.int32, (d, d), 0)
    wc = lax.broadcasted_iota(jnp.int32, (d, d), 1)
    wt = jnp.where(wr == wc, 1.0, 0.0).astype(jnp.float32)
    mom = jnp.zeros((d, d), jnp.float32)
    vel = jnp.zeros((d, d), jnp.float32)

    for i in range(_STEPS):
        z = jnp.dot(sn, wt, preferred_element_type=jnp.float32)
        zz = jnp.sum(z * z, axis=1, keepdims=True)
        izr = lax.rsqrt(zz)
        izr_row = jnp.reshape(izr, (1, n))
        ohzt = oht * izr_row
        csum = jnp.dot(ohzt, z, preferred_element_type=jnp.float32)
        zcs = lax.dot_general(z, csum, (((1,), (1,)), ((), ())),
                              preferred_element_type=jnp.float32)
        rs1 = jnp.sum(zcs, axis=1, keepdims=True)
        rs2 = jnp.sum(zcs * oh, axis=1, keepdims=True)
        rr = inv_ssum * (izr * (rs1 - 2.0 * rs2) + 1.0)
        coef = (inv_ssum - rr) * (izr * izr)
        tw = coef * z
        ohz = oh * izr
        pst = jnp.dot(snt, ohz, preferred_element_type=jnp.float32)
        u = jnp.sum(pst, axis=1, keepdims=True)
        m64t = inv_ssum * (u - 2.0 * pst)
        gwt = (
            jnp.dot(snt, tw, preferred_element_type=jnp.float32)
            + jnp.dot(m64t, csum, preferred_element_type=jnp.float32)
        )
        mom = _B1 * mom + (1.0 - _B1) * gwt
        vel = _B2 * vel + (1.0 - _B2) * gwt * gwt
        c2s = (1.0 / (1.0 - _B2 ** (i + 1))) ** 0.5
        a = _LR / (1.0 - _B1 ** (i + 1)) / c2s
        wt = wt - a * mom / (jnp.sqrt(vel) + _EPS / c2s)

    w = jnp.transpose(wt)
    dsupt = jnp.dot(w, snt, preferred_element_type=jnp.float32)
    w2t_ref[...] = -2.0 * wt
    dsupt_ref[...] = dsupt
    s2_ref[...] = jnp.sum(dsupt * dsupt, axis=0, keepdims=True)


def _knn_body(q_ref, w2t_ref, dsupt_ref, s2_ref, out_ref):
    q = q_ref[...]
    iqn = lax.rsqrt(jnp.sum(q * q, axis=1, keepdims=True))
    qn = q * iqn
    dq2 = jnp.dot(qn, w2t_ref[...],
                  preferred_element_type=jnp.float32)
    cross2 = jnp.dot(dq2, dsupt_ref[...],
                     preferred_element_type=jnp.float32)
    q2 = 0.25 * jnp.sum(dq2 * dq2, axis=1, keepdims=True)
    e = s2_ref[...] + cross2

    inf = jnp.float32(jnp.inf)
    nsup = e.shape[1]
    chunk = 128
    full = jnp.full((e.shape[0], chunk), inf, jnp.float32)
    a, b, c = full, full, full
    for k in range(nsup // chunk):
        x = e[:, k * chunk:(k + 1) * chunk]
        lo = jnp.minimum(a, x)
        hi = jnp.maximum(a, x)
        a = lo
        lo2 = jnp.minimum(b, hi)
        hi2 = jnp.maximum(b, hi)
        b = lo2
        c = jnp.minimum(c, hi2)
    cand = jnp.concatenate([a, b, c], axis=1)

    m1 = jnp.min(cand, axis=1, keepdims=True)
    c1 = jnp.sum(jnp.where(cand <= m1, 1.0, 0.0), axis=1, keepdims=True)
    masked = jnp.where(cand > m1, cand, inf)
    m2 = jnp.min(masked, axis=1, keepdims=True)
    c2 = jnp.sum(jnp.where(masked <= m2, 1.0, 0.0), axis=1, keepdims=True)
    masked2 = jnp.where(masked > m2, masked, inf)
    m3 = jnp.min(masked2, axis=1, keepdims=True)

    k1 = jnp.minimum(c1, 3.0)
    k2 = jnp.minimum(c2, 3.0 - k1)
    k3 = 3.0 - k1 - k2
    s1 = jnp.sqrt(jnp.maximum(m1 + q2, 0.0))
    s2v = jnp.sqrt(jnp.maximum(jnp.where(m2 < inf, m2, 0.0) + q2, 0.0))
    s3v = jnp.sqrt(jnp.maximum(jnp.where(m3 < inf, m3, 0.0) + q2, 0.0))
    sumd = k1 * s1 + k2 * s2v + k3 * s3v
    out_ref[...] = 1.0 - sumd * (1.0 / _K)


def kernel(support_features, support_labels, query_features, query_labels):
    n, d = support_features.shape
    nq = query_features.shape[0]
    lab = support_labels.astype(jnp.int32)
    lab_col = lab.reshape(n, 1)
    lab_row = lab.reshape(1, n)
    supt = support_features.T

    w2t, dsupt, s2_row = pl.pallas_call(
        _train_body,
        out_shape=[
            jax.ShapeDtypeStruct((d, d), jnp.float32),
            jax.ShapeDtypeStruct((d, n), jnp.float32),
            jax.ShapeDtypeStruct((1, n), jnp.float32),
        ],
    )(support_features, supt, lab_col, lab_row)

    bq = 1024
    grid = nq // bq
    out = pl.pallas_call(
        _knn_body,
        grid=(grid,),
        in_specs=[
            pl.BlockSpec((bq, d), lambda i: (i, 0)),
            pl.BlockSpec((d, d), lambda i: (0, 0)),
            pl.BlockSpec((d, n), lambda i: (0, 0)),
            pl.BlockSpec((1, n), lambda i: (0, 0)),
        ],
        out_specs=pl.BlockSpec((bq, 1), lambda i: (i, 0)),
        out_shape=jax.ShapeDtypeStruct((nq, 1), jnp.float32),
    )(query_features, w2t, dsupt, s2_row)
    return out.reshape(nq)


# speedup vs baseline: 1.0532x; 1.0532x over previous
import jax
import jax.numpy as jnp
from jax import lax
from jax.experimental import pallas as pl

_LR, _B1, _B2, _EPS = 1e-3, 0.9, 0.999, 1e-8
_STEPS = 10
_K = 3


def _train_body(sup_ref, lab_col_ref, w_ref, w2_ref, dsup_ref, s2_ref):
    sup = sup_ref[...]
    n, d = sup.shape
    isn = lax.rsqrt(jnp.sum(sup * sup, axis=1, keepdims=True))
    sn = sup * isn

    lc = lab_col_ref[...]
    ncls = 64
    onehot = jnp.where(
        lc == lax.broadcasted_iota(jnp.int32, (n, ncls), 1), 1.0, 0.0
    ).astype(jnp.float32)
    counts = jnp.sum(onehot, axis=0, keepdims=True)
    s_all = jnp.float32(n) * jnp.float32(n) - 2.0 * jnp.sum(counts * counts)
    inv_ssum = 2.0 / (s_all + jnp.float32(n))

    wr = lax.broadcasted_iota(jnp.int32, (d, d), 0)
    wc = lax.broadcasted_iota(jnp.int32, (d, d), 1)
    w = jnp.where(wr == wc, 1.0, 0.0).astype(jnp.float32)
    mom = jnp.zeros((d, d), jnp.float32)
    vel = jnp.zeros((d, d), jnp.float32)

    for i in range(_STEPS):
        z = lax.dot_general(sn, w, (((1,), (1,)), ((), ())),
                            preferred_element_type=jnp.float32)
        izr = lax.rsqrt(jnp.sum(z * z, axis=1, keepdims=True))
        t = z * izr
        csum = lax.dot_general(onehot, t, (((0,), (0,)), ((), ())),
                               preferred_element_type=jnp.float32)
        tcs = lax.dot_general(t, csum, (((1,), (1,)), ((), ())),
                              preferred_element_type=jnp.float32)
        rr = inv_ssum * (
            jnp.sum(tcs, axis=1, keepdims=True)
            - 2.0 * jnp.sum(tcs * onehot, axis=1, keepdims=True)
            + 1.0
        )
        ohz = onehot * izr
        ps = lax.dot_general(ohz, sn, (((0,), (0,)), ((), ())),
                             preferred_element_type=jnp.float32)
        m64 = inv_ssum * (jnp.sum(ps, axis=0, keepdims=True) - 2.0 * ps)
        tw = ((inv_ssum - rr) * izr) * t
        gw = (
            lax.dot_general(csum, m64, (((0,), (0,)), ((), ())),
                            preferred_element_type=jnp.float32)
            + lax.dot_general(tw, sn, (((0,), (0,)), ((), ())),
                              preferred_element_type=jnp.float32)
        )
        mom = _B1 * mom + (1.0 - _B1) * gw
        vel = _B2 * vel + (1.0 - _B2) * gw * gw
        c2s = (1.0 / (1.0 - _B2 ** (i + 1))) ** 0.5
        a = _LR / (1.0 - _B1 ** (i + 1)) / c2s
        w = w - a * mom / (jnp.sqrt(vel) + _EPS / c2s)

    dsup = lax.dot_general(sn, w, (((1,), (1,)), ((), ())),
                           preferred_element_type=jnp.float32)
    w_ref[...] = w
    w2_ref[...] = -2.0 * w
    dsup_ref[...] = dsup
    s2_ref[...] = jnp.sum(dsup * dsup, axis=1, keepdims=True)


def _knn_body(q_ref, w2_ref, dsup_ref, s2_ref, out_ref):
    q = q_ref[...]
    iqn = lax.rsqrt(jnp.sum(q * q, axis=1, keepdims=True))
    qn = q * iqn
    dq2 = lax.dot_general(qn, w2_ref[...], (((1,), (1,)), ((), ())),
                          preferred_element_type=jnp.float32)
    ds = dsup_ref[...]
    cross2 = lax.dot_general(dq2, ds, (((1,), (1,)), ((), ())),
                             preferred_element_type=jnp.float32)
    q2 = 0.25 * jnp.sum(dq2 * dq2, axis=1, keepdims=True)
    e = s2_ref[...] + cross2

    inf = jnp.float32(jnp.inf)
    nsup = e.shape[1]
    chunk = 128
    full = jnp.full((e.shape[0], chunk), inf, jnp.float32)
    a, b, c = full, full, full
    for k in range(nsup // chunk):
        x = e[:, k * chunk:(k + 1) * chunk]
        lo = jnp.minimum(a, x)
        hi = jnp.maximum(a, x)
        a = lo
        lo2 = jnp.minimum(b, hi)
        hi2 = jnp.maximum(b, hi)
        b = lo2
        c = jnp.minimum(c, hi2)
    cand = jnp.concatenate([a, b, c], axis=1)

    m1 = jnp.min(cand, axis=1, keepdims=True)
    c1 = jnp.sum(jnp.where(cand <= m1, 1.0, 0.0), axis=1, keepdims=True)
    masked = jnp.where(cand > m1, cand, inf)
    m2 = jnp.min(masked, axis=1, keepdims=True)
    c2 = jnp.sum(jnp.where(masked <= m2, 1.0, 0.0), axis=1, keepdims=True)
    masked2 = jnp.where(masked > m2, masked, inf)
    m3 = jnp.min(masked2, axis=1, keepdims=True)

    k1 = jnp.minimum(c1, 3.0)
    k2 = jnp.minimum(c2, 3.0 - k1)
    k3 = 3.0 - k1 - k2
    s1 = jnp.sqrt(jnp.maximum(m1 + q2, 0.0))
    s2v = jnp.sqrt(jnp.maximum(jnp.where(m2 < inf, m2, 0.0) + q2, 0.0))
    s3v = jnp.sqrt(jnp.maximum(jnp.where(m3 < inf, m3, 0.0) + q2, 0.0))
    sumd = k1 * s1 + k2 * s2v + k3 * s3v
    out_ref[...] = 1.0 - sumd * (1.0 / _K)


def kernel(support_features, support_labels, query_features, query_labels):
    n, d = support_features.shape
    nq = query_features.shape[0]
    lab_col = support_labels.astype(jnp.int32).reshape(n, 1)

    w, w2, dsup, s2 = pl.pallas_call(
        _train_body,
        out_shape=[
            jax.ShapeDtypeStruct((d, d), jnp.float32),
            jax.ShapeDtypeStruct((d, d), jnp.float32),
            jax.ShapeDtypeStruct((n, d), jnp.float32),
            jax.ShapeDtypeStruct((n, 1), jnp.float32),
        ],
    )(support_features, lab_col)

    s2_row = s2.reshape(1, n)
    bq = 512
    grid = nq // bq
    out = pl.pallas_call(
        _knn_body,
        grid=(grid,),
        in_specs=[
            pl.BlockSpec((bq, d), lambda i: (i, 0)),
            pl.BlockSpec((d, d), lambda i: (0, 0)),
            pl.BlockSpec((n, d), lambda i: (0, 0)),
            pl.BlockSpec((1, n), lambda i: (0, 0)),
        ],
        out_specs=pl.BlockSpec((bq, 1), lambda i: (i, 0)),
        out_shape=jax.ShapeDtypeStruct((nq, 1), jnp.float32),
    )(query_features, w2, dsup, s2_row)
    return out.reshape(nq)


# speedup vs baseline: 1.0551x; 1.0018x over previous
import jax
import jax.numpy as jnp
from jax import lax
from jax.experimental import pallas as pl

_LR, _B1, _B2, _EPS = 1e-3, 0.9, 0.999, 1e-8
_STEPS = 10
_K = 3


def _train_body(sup_ref, lab_col_ref, w_ref, w2_ref, dsup_ref, s2_ref):
    sup = sup_ref[...]
    n, d = sup.shape
    isn = lax.rsqrt(jnp.sum(sup * sup, axis=1, keepdims=True))
    sn = sup * isn

    lc = lab_col_ref[...]
    ncls = 64
    onehot = jnp.where(
        lc == lax.broadcasted_iota(jnp.int32, (n, ncls), 1), 1.0, 0.0
    ).astype(jnp.float32)
    counts = jnp.sum(onehot, axis=0, keepdims=True)
    s_all = jnp.float32(n) * jnp.float32(n) - 2.0 * jnp.sum(counts * counts)
    inv_ssum = 2.0 / (s_all + jnp.float32(n))

    wr = lax.broadcasted_iota(jnp.int32, (d, d), 0)
    wc = lax.broadcasted_iota(jnp.int32, (d, d), 1)
    w = jnp.where(wr == wc, 1.0, 0.0).astype(jnp.float32)
    mom = jnp.zeros((d, d), jnp.float32)
    vel = jnp.zeros((d, d), jnp.float32)

    for i in range(_STEPS):
        z = lax.dot_general(sn, w, (((1,), (1,)), ((), ())),
                            preferred_element_type=jnp.float32)
        izr = lax.rsqrt(jnp.sum(z * z, axis=1, keepdims=True))
        t = z * izr
        csum = lax.dot_general(onehot, t, (((0,), (0,)), ((), ())),
                               preferred_element_type=jnp.float32)
        tcs = lax.dot_general(t, csum, (((1,), (1,)), ((), ())),
                              preferred_element_type=jnp.float32)
        rr = inv_ssum * (
            jnp.sum(tcs, axis=1, keepdims=True)
            - 2.0 * jnp.sum(tcs * onehot, axis=1, keepdims=True)
            + 1.0
        )
        ohz = onehot * izr
        ps = lax.dot_general(ohz, sn, (((0,), (0,)), ((), ())),
                             preferred_element_type=jnp.float32)
        m64 = inv_ssum * (jnp.sum(ps, axis=0, keepdims=True) - 2.0 * ps)
        tw = ((inv_ssum - rr) * izr) * t
        gw = (
            lax.dot_general(csum, m64, (((0,), (0,)), ((), ())),
                            preferred_element_type=jnp.float32)
            + lax.dot_general(tw, sn, (((0,), (0,)), ((), ())),
                              preferred_element_type=jnp.float32)
        )
        mom = _B1 * mom + (1.0 - _B1) * gw
        vel = _B2 * vel + (1.0 - _B2) * gw * gw
        c2s = (1.0 / (1.0 - _B2 ** (i + 1))) ** 0.5
        a = _LR / (1.0 - _B1 ** (i + 1)) / c2s
        w = w - a * mom / (jnp.sqrt(vel) + _EPS / c2s)

    dsup = lax.dot_general(sn, w, (((1,), (1,)), ((), ())),
                           preferred_element_type=jnp.float32)
    w_ref[...] = w
    w2_ref[...] = -2.0 * w
    dsup_ref[...] = dsup
    s2_ref[...] = jnp.sum(dsup * dsup, axis=1, keepdims=True)


def _knn_body(q_ref, w2_ref, dsup_ref, s2_ref, out_ref):
    q = q_ref[...]
    iqn = lax.rsqrt(jnp.sum(q * q, axis=1, keepdims=True))
    qn = q * iqn
    dq2 = lax.dot_general(qn, w2_ref[...], (((1,), (1,)), ((), ())),
                          preferred_element_type=jnp.float32)
    ds = dsup_ref[...]
    cross2 = lax.dot_general(dq2, ds, (((1,), (1,)), ((), ())),
                             preferred_element_type=jnp.float32)
    q2 = 0.25 * jnp.sum(dq2 * dq2, axis=1, keepdims=True)
    e = s2_ref[...] + cross2

    inf = jnp.float32(jnp.inf)
    nsup = e.shape[1]
    chunk = 128
    full = jnp.full((e.shape[0], chunk), inf, jnp.float32)
    a, b, c = full, full, full
    for k in range(nsup // chunk):
        x = e[:, k * chunk:(k + 1) * chunk]
        lo = jnp.minimum(a, x)
        hi = jnp.maximum(a, x)
        a = lo
        lo2 = jnp.minimum(b, hi)
        hi2 = jnp.maximum(b, hi)
        b = lo2
        c = jnp.minimum(c, hi2)
    cand = jnp.concatenate([a, b, c], axis=1)

    m1 = jnp.min(cand, axis=1, keepdims=True)
    c1 = jnp.sum(jnp.where(cand <= m1, 1.0, 0.0), axis=1, keepdims=True)
    masked = jnp.where(cand > m1, cand, inf)
    m2 = jnp.min(masked, axis=1, keepdims=True)
    c2 = jnp.sum(jnp.where(masked <= m2, 1.0, 0.0), axis=1, keepdims=True)
    masked2 = jnp.where(masked > m2, masked, inf)
    m3 = jnp.min(masked2, axis=1, keepdims=True)

    k1 = jnp.minimum(c1, 3.0)
    k2 = jnp.minimum(c2, 3.0 - k1)
    k3 = 3.0 - k1 - k2
    s1 = jnp.sqrt(jnp.maximum(m1 + q2, 0.0))
    s2v = jnp.sqrt(jnp.maximum(jnp.where(m2 < inf, m2, 0.0) + q2, 0.0))
    s3v = jnp.sqrt(jnp.maximum(jnp.where(m3 < inf, m3, 0.0) + q2, 0.0))
    sumd = k1 * s1 + k2 * s2v + k3 * s3v
    out_ref[...] = 1.0 - sumd * (1.0 / _K)


def kernel(support_features, support_labels, query_features, query_labels):
    n, d = support_features.shape
    nq = query_features.shape[0]
    lab_col = support_labels.astype(jnp.int32).reshape(n, 1)

    w, w2, dsup, s2 = pl.pallas_call(
        _train_body,
        out_shape=[
            jax.ShapeDtypeStruct((d, d), jnp.float32),
            jax.ShapeDtypeStruct((d, d), jnp.float32),
            jax.ShapeDtypeStruct((n, d), jnp.float32),
            jax.ShapeDtypeStruct((n, 1), jnp.float32),
        ],
    )(support_features, lab_col)

    s2_row = s2.reshape(1, n)
    bq = 2048
    grid = nq // bq
    out = pl.pallas_call(
        _knn_body,
        grid=(grid,),
        in_specs=[
            pl.BlockSpec((bq, d), lambda i: (i, 0)),
            pl.BlockSpec((d, d), lambda i: (0, 0)),
            pl.BlockSpec((n, d), lambda i: (0, 0)),
            pl.BlockSpec((1, n), lambda i: (0, 0)),
        ],
        out_specs=pl.BlockSpec((bq, 1), lambda i: (i, 0)),
        out_shape=jax.ShapeDtypeStruct((nq, 1), jnp.float32),
    )(query_features, w2, dsup, s2_row)
    return out.reshape(nq)


# speedup vs baseline: 1.0725x; 1.0165x over previous
import jax
import jax.numpy as jnp
from jax import lax
from jax.experimental import pallas as pl

_LR, _B1, _B2, _EPS = 1e-3, 0.9, 0.999, 1e-8
_STEPS = 10
_K = 3


def _train_body(sup_ref, lab_col_ref, w_ref, w2_ref, dsup_ref, s2_ref):
    sup = sup_ref[...]
    n, d = sup.shape
    isn = lax.rsqrt(jnp.sum(sup * sup, axis=1, keepdims=True))
    sn = sup * isn

    lc = lab_col_ref[...]
    ncls = 64
    onehot = jnp.where(
        lc == lax.broadcasted_iota(jnp.int32, (n, ncls), 1), 1.0, 0.0
    ).astype(jnp.float32)
    counts = jnp.sum(onehot, axis=0, keepdims=True)
    s_all = jnp.float32(n) * jnp.float32(n) - 2.0 * jnp.sum(counts * counts)
    inv_ssum = 2.0 / (s_all + jnp.float32(n))

    wr = lax.broadcasted_iota(jnp.int32, (d, d), 0)
    wc = lax.broadcasted_iota(jnp.int32, (d, d), 1)
    w = jnp.where(wr == wc, 1.0, 0.0).astype(jnp.float32)
    mom = jnp.zeros((d, d), jnp.float32)
    vel = jnp.zeros((d, d), jnp.float32)

    for i in range(_STEPS):
        z = lax.dot_general(sn, w, (((1,), (1,)), ((), ())),
                            preferred_element_type=jnp.float32)
        izr = lax.rsqrt(jnp.sum(z * z, axis=1, keepdims=True))
        t = z * izr
        csum = lax.dot_general(onehot, t, (((0,), (0,)), ((), ())),
                               preferred_element_type=jnp.float32)
        tcs = lax.dot_general(t, csum, (((1,), (1,)), ((), ())),
                              preferred_element_type=jnp.float32)
        rr = inv_ssum * (
            jnp.sum(tcs, axis=1, keepdims=True)
            - 2.0 * jnp.sum(tcs * onehot, axis=1, keepdims=True)
            + 1.0
        )
        ohz = onehot * izr
        ps = lax.dot_general(ohz, sn, (((0,), (0,)), ((), ())),
                             preferred_element_type=jnp.float32)
        m64 = inv_ssum * (jnp.sum(ps, axis=0, keepdims=True) - 2.0 * ps)
        tw = ((inv_ssum - rr) * izr) * t
        gw = (
            lax.dot_general(csum, m64, (((0,), (0,)), ((), ())),
                            preferred_element_type=jnp.float32)
            + lax.dot_general(tw, sn, (((0,), (0,)), ((), ())),
                              preferred_element_type=jnp.float32)
        )
        mom = _B1 * mom + (1.0 - _B1) * gw
        vel = _B2 * vel + (1.0 - _B2) * gw * gw
        c2s = (1.0 / (1.0 - _B2 ** (i + 1))) ** 0.5
        a = _LR / (1.0 - _B1 ** (i + 1)) / c2s
        w = w - a * mom / (jnp.sqrt(vel) + _EPS / c2s)

    dsup = lax.dot_general(sn, w, (((1,), (1,)), ((), ())),
                           preferred_element_type=jnp.float32)
    w_ref[...] = w
    w2_ref[...] = -2.0 * w
    dsup_ref[...] = dsup
    s2_ref[...] = jnp.sum(dsup * dsup, axis=1, keepdims=True)


def _knn_body(q_ref, w2_ref, dsup_ref, s2_ref, out_ref):
    q = q_ref[...]
    iqn = lax.rsqrt(jnp.sum(q * q, axis=1, keepdims=True))
    qn = q * iqn
    dq2 = lax.dot_general(qn, w2_ref[...], (((1,), (1,)), ((), ())),
                          preferred_element_type=jnp.float32)
    ds = dsup_ref[...]
    cross2 = lax.dot_general(dq2, ds, (((1,), (1,)), ((), ())),
                             preferred_element_type=jnp.float32)
    q2 = 0.25 * jnp.sum(dq2 * dq2, axis=1, keepdims=True)
    e = s2_ref[...] + cross2

    inf = jnp.float32(jnp.inf)
    nsup = e.shape[1]
    chunk = 128
    full = jnp.full((e.shape[0], chunk), inf, jnp.float32)
    a, b, c = full, full, full
    for k in range(nsup // chunk):
        x = e[:, k * chunk:(k + 1) * chunk]
        lo = jnp.minimum(a, x)
        hi = jnp.maximum(a, x)
        a = lo
        lo2 = jnp.minimum(b, hi)
        hi2 = jnp.maximum(b, hi)
        b = lo2
        c = jnp.minimum(c, hi2)
    cand = jnp.concatenate([a, b, c], axis=1)

    m1 = jnp.min(cand, axis=1, keepdims=True)
    c1 = jnp.sum(jnp.where(cand <= m1, 1.0, 0.0), axis=1, keepdims=True)
    masked = jnp.where(cand > m1, cand, inf)
    m2 = jnp.min(masked, axis=1, keepdims=True)
    c2 = jnp.sum(jnp.where(masked <= m2, 1.0, 0.0), axis=1, keepdims=True)
    masked2 = jnp.where(masked > m2, masked, inf)
    m3 = jnp.min(masked2, axis=1, keepdims=True)

    k1 = jnp.minimum(c1, 3.0)
    k2 = jnp.minimum(c2, 3.0 - k1)
    k3 = 3.0 - k1 - k2
    s1 = jnp.sqrt(jnp.maximum(m1 + q2, 0.0))
    s2v = jnp.sqrt(jnp.maximum(jnp.where(m2 < inf, m2, 0.0) + q2, 0.0))
    s3v = jnp.sqrt(jnp.maximum(jnp.where(m3 < inf, m3, 0.0) + q2, 0.0))
    sumd = k1 * s1 + k2 * s2v + k3 * s3v
    out_ref[...] = 1.0 - sumd * (1.0 / _K)


def kernel(support_features, support_labels, query_features, query_labels):
    n, d = support_features.shape
    nq = query_features.shape[0]
    lab_col = support_labels.astype(jnp.int32).reshape(n, 1)

    w, w2, dsup, s2 = pl.pallas_call(
        _train_body,
        out_shape=[
            jax.ShapeDtypeStruct((d, d), jnp.float32),
            jax.ShapeDtypeStruct((d, d), jnp.float32),
            jax.ShapeDtypeStruct((n, d), jnp.float32),
            jax.ShapeDtypeStruct((n, 1), jnp.float32),
        ],
    )(support_features, lab_col)

    s2_row = s2.reshape(1, n)
    bq = 1024
    grid = nq // bq
    out = pl.pallas_call(
        _knn_body,
        grid=(grid,),
        in_specs=[
            pl.BlockSpec((bq, d), lambda i: (i, 0)),
            pl.BlockSpec((d, d), lambda i: (0, 0)),
            pl.BlockSpec((n, d), lambda i: (0, 0)),
            pl.BlockSpec((1, n), lambda i: (0, 0)),
        ],
        out_specs=pl.BlockSpec((bq, 1), lambda i: (i, 0)),
        out_shape=jax.ShapeDtypeStruct((nq, 1), jnp.float32),
    )(query_features, w2, dsup, s2_row)
    return out.reshape(nq)
